# baseline (speedup 1.0000x reference)
import functools

import jax
import jax.numpy as jnp
from jax.experimental import pallas as pl
from jax.experimental.pallas import tpu as pltpu


def _mm_body(a_ref, b_ref, o_ref, *, sub):
    ms = pl.program_id(2)
    a_blk = a_ref[0, pl.ds(ms * sub, sub), :]
    o_ref[0] = jax.lax.dot_general(
        a_blk, b_ref[0],
        dimension_numbers=(((1,), (0,)), ((), ())),
        preferred_element_type=jnp.float32,
        precision=jax.lax.Precision.DEFAULT,
    )


@functools.partial(jax.jit, static_argnames=("tm", "nsub"))
def _batched_mm(a3, b3, tm=1024, nsub=4):
    nb, m, k = a3.shape
    n = b3.shape[-1]
    sub = tm // nsub
    return pl.pallas_call(
        functools.partial(_mm_body, sub=sub),
        grid=(nb, m // tm, nsub),
        in_specs=[
            pl.BlockSpec((1, tm, k), lambda b, i, ms: (b, i, 0)),
            pl.BlockSpec((1, k, n), lambda b, i, ms: (b, 0, 0)),
        ],
        out_specs=pl.BlockSpec((1, sub, n),
                               lambda b, i, ms, ns=nsub: (b, i * ns + ms, 0)),
        out_shape=jax.ShapeDtypeStruct((nb, m, n), jnp.float32),
    )(a3, b3)


def kernel(a, b):
    B1, B2, M, K = a.shape
    N = b.shape[-1]
    a3 = a.reshape(B1 * B2, M, K)
    b3 = b.reshape(B1 * B2, K, N)
    out = _batched_mm(a3, b3, tm=min(1024, M))
    return out.reshape(B1, B2, M, N)


# speedup vs baseline: 1.7328x; 1.7328x over previous
import functools

import jax
import jax.numpy as jnp
from jax.experimental import pallas as pl


def _mm_body(a0_ref, a1_ref, b0_ref, b1_ref, o_ref):
    dn = (((1,), (0,)), ((), ()))
    acc = jax.lax.dot_general(
        a0_ref[0], b0_ref[0], dimension_numbers=dn,
        preferred_element_type=jnp.float32,
        precision=jax.lax.Precision.DEFAULT,
    )
    acc += jax.lax.dot_general(
        a1_ref[0], b1_ref[0], dimension_numbers=dn,
        preferred_element_type=jnp.float32,
        precision=jax.lax.Precision.DEFAULT,
    )
    o_ref[0] = acc


@functools.partial(jax.jit, static_argnames=("tm",))
def _batched_mm(a3, b3, tm=2048):
    nb, m, k = a3.shape
    n = b3.shape[-1]
    hk = k // 2
    return pl.pallas_call(
        _mm_body,
        grid=(nb, m // tm),
        in_specs=[
            pl.BlockSpec((1, tm, hk), lambda b, i: (b, i, 0)),
            pl.BlockSpec((1, tm, hk), lambda b, i: (b, i, 1)),
            pl.BlockSpec((1, hk, n), lambda b, i: (b, 0, 0)),
            pl.BlockSpec((1, hk, n), lambda b, i: (b, 1, 0)),
        ],
        out_specs=pl.BlockSpec((1, tm, n), lambda b, i: (b, i, 0)),
        out_shape=jax.ShapeDtypeStruct((nb, m, n), jnp.float32),
    )(a3, a3, b3, b3)


def kernel(a, b):
    B1, B2, M, K = a.shape
    N = b.shape[-1]
    a3 = a.reshape(B1 * B2, M, K)
    b3 = b.reshape(B1 * B2, K, N)
    out = _batched_mm(a3, b3, tm=min(2048, M))
    return out.reshape(B1, B2, M, N)


# speedup vs baseline: 1.7624x; 1.0171x over previous
import functools

import jax
import jax.numpy as jnp
from jax.experimental import pallas as pl


def _mm_body(a_ref, b_ref, o_ref):
    o_ref[0] = jax.lax.dot_general(
        a_ref[0], b_ref[0],
        dimension_numbers=(((1,), (0,)), ((), ())),
        preferred_element_type=jnp.float32,
        precision=jax.lax.Precision.DEFAULT,
    )


@functools.partial(jax.jit, static_argnames=("tm",))
def _batched_mm(a3, b3, tm=1024):
    nb, m, k = a3.shape
    n = b3.shape[-1]
    mt = m // tm
    return pl.pallas_call(
        _mm_body,
        grid=(nb * mt,),
        in_specs=[
            pl.BlockSpec((1, tm, k), lambda g, mt=mt: (g // mt, g % mt, 0)),
            pl.BlockSpec((1, k, n), lambda g, mt=mt: (g // mt, 0, 0)),
        ],
        out_specs=pl.BlockSpec((1, tm, n), lambda g, mt=mt: (g // mt, g % mt, 0)),
        out_shape=jax.ShapeDtypeStruct((nb, m, n), jnp.float32),
    )(a3, b3)


def kernel(a, b):
    B1, B2, M, K = a.shape
    N = b.shape[-1]
    a3 = a.reshape(B1 * B2, M, K)
    b3 = b.reshape(B1 * B2, K, N)
    out = _batched_mm(a3, b3, tm=min(1024, M))
    return out.reshape(B1, B2, M, N)


# speedup vs baseline: 1.7972x; 1.0198x over previous
import functools

import jax
import jax.numpy as jnp
from jax.experimental import pallas as pl
from jax.experimental.pallas import tpu as pltpu


def _mm_body(a_hbm, b_hbm, o_hbm, abuf, bbuf, obuf, asem, bsem, osem,
             *, nch, tot, ch, depth):
    g = pl.program_id(0)
    batch = g // nch
    chunk = g % nch
    row = chunk * ch

    def start_a(c):
        pltpu.make_async_copy(
            a_hbm.at[c // nch, pl.ds((c % nch) * ch, ch), :],
            abuf.at[c % depth],
            asem.at[c % depth],
        ).start()

    def start_b(k):
        pltpu.make_async_copy(b_hbm.at[k], bbuf.at[k % 2], bsem.at[k % 2]).start()

    @pl.when(g == 0)
    def _prologue():
        start_b(0)
        for c in range(min(depth - 1, tot)):
            start_a(c)

    @pl.when((chunk == 0) & (batch + 1 < o_hbm.shape[0]))
    def _next_b():
        start_b(batch + 1)

    @pl.when(chunk == 0)
    def _wait_b():
        pltpu.make_async_copy(b_hbm.at[batch], bbuf.at[batch % 2],
                              bsem.at[batch % 2]).wait()

    @pl.when(g >= 2)
    def _wait_out_slot():
        pltpu.make_async_copy(obuf.at[g % 2],
                              o_hbm.at[0, pl.ds(0, ch), :], osem.at[g % 2]).wait()

    pltpu.make_async_copy(a_hbm.at[batch, pl.ds(row, ch), :],
                          abuf.at[g % depth], asem.at[g % depth]).wait()

    obuf[g % 2] = jax.lax.dot_general(
        abuf[g % depth], bbuf[batch % 2],
        dimension_numbers=(((1,), (0,)), ((), ())),
        preferred_element_type=jnp.float32,
        precision=jax.lax.Precision.DEFAULT,
    )

    pltpu.make_async_copy(obuf.at[g % 2], o_hbm.at[batch, pl.ds(row, ch), :],
                          osem.at[g % 2]).start()

    @pl.when(g + depth - 1 < tot)
    def _next_a():
        start_a(g + depth - 1)

    @pl.when(g == tot - 1)
    def _epilogue():
        pltpu.make_async_copy(obuf.at[(g - 1) % 2],
                              o_hbm.at[0, pl.ds(0, ch), :], osem.at[(g - 1) % 2]).wait()
        pltpu.make_async_copy(obuf.at[g % 2],
                              o_hbm.at[0, pl.ds(0, ch), :], osem.at[g % 2]).wait()


@functools.partial(jax.jit, static_argnames=("ch", "depth"))
def _batched_mm(a3, b3, ch=512, depth=4):
    nb, m, k = a3.shape
    n = b3.shape[-1]
    nch = m // ch
    tot = nb * nch
    body = functools.partial(_mm_body, nch=nch, tot=tot, ch=ch, depth=depth)
    return pl.pallas_call(
        body,
        grid=(tot,),
        in_specs=[
            pl.BlockSpec(memory_space=pltpu.HBM),
            pl.BlockSpec(memory_space=pltpu.HBM),
        ],
        out_specs=pl.BlockSpec(memory_space=pltpu.HBM),
        out_shape=jax.ShapeDtypeStruct((nb, m, n), jnp.float32),
        scratch_shapes=[
            pltpu.VMEM((depth, ch, k), jnp.float32),
            pltpu.VMEM((2, k, n), jnp.float32),
            pltpu.VMEM((2, ch, n), jnp.float32),
            pltpu.SemaphoreType.DMA((depth,)),
            pltpu.SemaphoreType.DMA((2,)),
            pltpu.SemaphoreType.DMA((2,)),
        ],
        compiler_params=pltpu.CompilerParams(
            dimension_semantics=("arbitrary",),
        ),
    )(a3, b3)


def kernel(a, b):
    B1, B2, M, K = a.shape
    N = b.shape[-1]
    a3 = a.reshape(B1 * B2, M, K)
    b3 = b.reshape(B1 * B2, K, N)
    out = _batched_mm(a3, b3, ch=min(512, M), depth=4)
    return out.reshape(B1, B2, M, N)
